# baseline (speedup 1.0000x reference)
import functools

import jax
import jax.numpy as jnp
from jax import lax
from jax.experimental import pallas as pl
from jax.experimental.pallas import tpu as pltpu


def _mm_body(x_ref, w_ref, b_ref, o_ref, *, relu):
    y = jnp.dot(x_ref[...], w_ref[...], preferred_element_type=jnp.float32)
    y = y + b_ref[...]
    if relu:
        y = jnp.maximum(y, 0.0)
    o_ref[...] = y


def _matmul_bias(x, W, b, relu=False, block_m=2000):
    M, K = x.shape
    _, N = W.shape
    grid = (M // block_m,)
    return pl.pallas_call(
        functools.partial(_mm_body, relu=relu),
        grid=grid,
        in_specs=[
            pl.BlockSpec((block_m, K), lambda i: (i, 0)),
            pl.BlockSpec((K, N), lambda i: (0, 0)),
            pl.BlockSpec((1, N), lambda i: (0, 0)),
        ],
        out_specs=pl.BlockSpec((block_m, N), lambda i: (i, 0)),
        out_shape=jax.ShapeDtypeStruct((M, N), jnp.float32),
    )(x, W, b.reshape(1, -1))


def _head_body(h_ref, nv_ref, wl1_ref, bl1_ref, wl2_ref, bl2_ref, o_ref):
    h = h_ref[...]
    nv = nv_ref[...]
    gmax = jnp.max(jnp.where(nv > 0, h, -jnp.inf), axis=0, keepdims=True)
    gsum = jnp.sum(h, axis=0, keepdims=True)
    cnt = jnp.maximum(jnp.sum(nv), 1.0)
    g = jnp.concatenate([gmax, gsum / cnt], axis=1)
    z = jnp.maximum(jnp.dot(g, wl1_ref[...], preferred_element_type=jnp.float32)
                    + bl1_ref[...], 0.0)
    logits = jnp.dot(z, wl2_ref[...], preferred_element_type=jnp.float32) + bl2_ref[...]
    o_ref[...] = jax.nn.log_softmax(logits, axis=-1)


def _head(h, nv_f32, Wl1, bl1, Wl2, bl2):
    M, _ = h.shape
    return pl.pallas_call(
        _head_body,
        out_shape=jax.ShapeDtypeStruct((1, Wl2.shape[1]), jnp.float32),
    )(h, nv_f32.reshape(M, 1), Wl1, bl1.reshape(1, -1), Wl2, bl2.reshape(1, -1))


def _conv_aggregate(h, src, dst, node_valid, edge_valid):
    N = h.shape[0]
    deg = jnp.zeros((N,), h.dtype).at[dst].add(
        jnp.where(edge_valid, 1.0, 0.0).astype(h.dtype))
    deg = deg + jnp.where(node_valid, 1.0, 0.0)
    dinv = jnp.where(deg > 0, lax.rsqrt(deg), 0.0)
    norm = jnp.where(edge_valid, dinv[src] * dinv[dst], 0.0)[:, None]
    out = jnp.zeros_like(h).at[dst].add(h[src] * norm)
    out = out + h * jnp.where(node_valid, dinv * dinv, 0.0)[:, None]
    return out


def _kmis(score, src, dst, N, node_valid, edge_valid):
    s = score.reshape(-1)
    s_eff = jnp.where(node_valid, s, -jnp.inf)
    perm = jnp.argsort(-s_eff)
    rank = jnp.zeros((N,), jnp.int32).at[perm].set(jnp.arange(N, dtype=jnp.int32))
    ss = jnp.concatenate([src, dst])
    dd = jnp.concatenate([dst, src])
    em = jnp.concatenate([edge_valid, edge_valid])
    BIG = jnp.int32(N)

    def cond(state):
        _, mask = state
        return jnp.any(mask)

    def body(state):
        mis, mask = state
        r = jnp.where(mask, rank, BIG)
        nmin = jnp.full((N,), BIG, jnp.int32).at[dd].min(jnp.where(em, r[ss], BIG))
        local = mask & (r <= nmin)
        mis = mis | local
        nb = jnp.zeros((N,), jnp.int32).at[dd].max(
            jnp.where(em, local[ss].astype(jnp.int32), 0)) > 0
        mask = mask & (~local) & (~nb)
        return mis, mask

    mis, _ = lax.while_loop(cond, body, (jnp.zeros((N,), bool), node_valid))
    r_mis = jnp.where(mis, rank, BIG)
    cand = jnp.full((N,), BIG, jnp.int32).at[dd].min(jnp.where(em, r_mis[ss], BIG))
    cand = jnp.minimum(cand, r_mis)
    cluster_node = perm[jnp.clip(cand, 0, N - 1)]
    Nc = jnp.sum(mis).astype(jnp.int32)
    new_id = jnp.where(mis, jnp.cumsum(mis.astype(jnp.int32)) - 1, 0)
    cluster = new_id[cluster_node]
    cu = cluster[src]
    cv = cluster[dst]
    keep = (cu != cv) & edge_valid
    SENT = jnp.int32(jnp.iinfo(jnp.int32).max)
    key = jnp.sort(jnp.where(keep, cu * Nc + cv, SENT))
    uniq = (key < SENT) & jnp.concatenate(
        [jnp.ones((1,), bool), key[1:] != key[:-1]])
    den = jnp.maximum(Nc, 1)
    new_src = jnp.where(uniq, key // den, 0).astype(jnp.int32)
    new_dst = jnp.where(uniq, key % den, 0).astype(jnp.int32)
    return mis, new_id, Nc, new_src, new_dst, uniq


def kernel(x, edge_index, batch, W1, b1, ws1, bs1, W2, b2, ws2, bs2,
           W3, b3, Wl1, bl1, Wl2, bl2):
    src = edge_index[0]
    dst = edge_index[1]
    N = x.shape[0]
    E = src.shape[0]
    ones_n = jnp.ones((N,), bool)
    ones_e = jnp.ones((E,), bool)

    xw = _matmul_bias(x, W1, jnp.zeros_like(b1))
    h = jnp.maximum(_conv_aggregate(xw, src, dst, ones_n, ones_e) + b1[None, :], 0.0)
    s1 = h @ ws1 + bs1

    mis1, nid1, Nc1, src1, dst1, ev2 = _kmis(s1, src, dst, N, ones_n, ones_e)
    idx1 = jnp.where(mis1, nid1, N)
    val1 = h * s1
    h = jnp.zeros_like(val1).at[idx1].set(val1, mode="drop")
    bt2 = jnp.zeros((N,), batch.dtype).at[idx1].set(batch, mode="drop")
    nv2 = jnp.arange(N, dtype=jnp.int32) < Nc1

    hw = _matmul_bias(h, W2, jnp.zeros_like(b2))
    agg = _conv_aggregate(hw, src1, dst1, nv2, ev2)
    h = jnp.where(nv2[:, None], jnp.maximum(agg + b2[None, :], 0.0), 0.0)
    s2 = h @ ws2 + bs2

    mis2, nid2, Nc2, src2, dst2, ev3 = _kmis(s2, src1, dst1, N, nv2, ev2)
    idx2 = jnp.where(mis2, nid2, N)
    val2 = h * s2
    h = jnp.zeros_like(val2).at[idx2].set(val2, mode="drop")
    bt3 = jnp.zeros((N,), bt2.dtype).at[idx2].set(bt2, mode="drop")
    nv3 = jnp.arange(N, dtype=jnp.int32) < Nc2

    hw = _matmul_bias(h, W3, jnp.zeros_like(b3))
    agg = _conv_aggregate(hw, src2, dst2, nv3, ev3)
    h = jnp.where(nv3[:, None], jnp.maximum(agg + b3[None, :], 0.0), 0.0)

    return _head(h, nv3.astype(jnp.float32), Wl1, bl1, Wl2, bl2)
